# baseline (speedup 1.0000x reference)
import functools

import jax
import jax.numpy as jnp
from jax import lax
from jax.experimental import pallas as pl
from jax.experimental.pallas import tpu as pltpu
from jax.experimental.pallas import tpu_sc as plsc

_N = 10000
_E = 320000
_H = 128
_NG = 16
_OUT = 10

_NTILES = 32
_EPAD = 327680
_CHUNK = 128
_HH = 64
_CPT = _EPAD // 16
_CR = 256
_NSC = _CPT // _CR
_NB = 4
_NGRP = _NSC // _NB
_IPG = _NB
_NP = 10240
_STRIPE = _NP // 16

_BLK = 1000
_NBLK = _N // _BLK


def _sc_body(x_hbm, src2_hbm, dst2_hbm, zeros_hbm, out_hbm,
             sidx0, sidx1, didx0, didx1, rows, acc,
             sg0, sg1, sg2, sg3, ss0, ss1, ss2, ss3,
             sis0, sid0, sis1, sid1):
  c = lax.axis_index("c")
  s = lax.axis_index("s")
  base = s * _NSC
  sem_g = (sg0, sg1, sg2, sg3)
  sem_s = (ss0, ss1, ss2, ss3)

  pltpu.sync_copy(zeros_hbm, rows.at[0, pl.ds(0, _CHUNK)])
  for j in range(_STRIPE // _CHUNK):
    pltpu.sync_copy(rows.at[0, pl.ds(0, _CHUNK)],
                    acc.at[pl.ds(s * _STRIPE + j * _CHUNK, _CHUNK)])
  plsc.subcore_barrier()

  pltpu.async_copy(src2_hbm.at[c, pl.ds(base, _IPG)], sidx0, sis0)
  pltpu.async_copy(dst2_hbm.at[pl.ds(base, _IPG)], didx0, sid0)
  pltpu.make_async_copy(src2_hbm.at[c, pl.ds(base, _IPG)], sidx0, sis0).wait()
  pltpu.make_async_copy(dst2_hbm.at[pl.ds(base, _IPG)], didx0, sid0).wait()
  for b in range(_NB):
    pltpu.async_copy(x_hbm.at[sidx0.at[b]], rows.at[b], sem_g[b])

  def _half(g, sidx_c, didx_c, sidx_n, didx_n, sis_n, sid_n, last):
    for b in range(_NB):
      pltpu.make_async_copy(x_hbm.at[sidx_c.at[0]],
                            rows.at[b], sem_g[b]).wait()
      pltpu.async_copy(rows.at[b], acc.at[didx_c.at[b]],
                       sem_s[b], add=True)

    @pl.when(jnp.logical_not(last))
    def _():
      pltpu.async_copy(src2_hbm.at[c, pl.ds(base + (g + 1) * _IPG, _IPG)],
                       sidx_n, sis_n)
      pltpu.async_copy(dst2_hbm.at[pl.ds(base + (g + 1) * _IPG, _IPG)],
                       didx_n, sid_n)

    for b in range(_NB):
      pltpu.make_async_copy(rows.at[b], acc.at[didx_c.at[0]],
                            sem_s[b]).wait()
      if b == 0:
        @pl.when(jnp.logical_not(last))
        def _():
          pltpu.make_async_copy(
              src2_hbm.at[c, pl.ds(base, _IPG)], sidx_n, sis_n).wait()
          pltpu.make_async_copy(
              dst2_hbm.at[pl.ds(base, _IPG)], didx_n, sid_n).wait()

      @pl.when(jnp.logical_not(last))
      def _():
        pltpu.async_copy(x_hbm.at[sidx_n.at[b]],
                         rows.at[b], sem_g[b])

  def body(t, carry):
    g0 = 2 * t
    _half(g0, sidx0, didx0, sidx1, didx1, sis1, sid1, g0 >= _NGRP - 1)
    _half(g0 + 1, sidx1, didx1, sidx0, didx0, sis0, sid0, g0 + 1 >= _NGRP - 1)
    return carry

  lax.fori_loop(0, _NGRP // 2, body, 0)
  plsc.subcore_barrier()

  pltpu.sync_copy(acc.at[pl.ds(s * _STRIPE, _STRIPE)],
                  out_hbm.at[c, pl.ds(s * _STRIPE, _STRIPE)])


def _sc_scatter(xs, src2, dst2, zeros_blk):
  k = pl.kernel(
      _sc_body,
      out_type=jax.ShapeDtypeStruct((2, _NP, _HH), jnp.float32),
      mesh=plsc.VectorSubcoreMesh(core_axis_name="c", subcore_axis_name="s"),
      scratch_types=[
          pltpu.VMEM((_IPG, _CR), jnp.int32),
          pltpu.VMEM((_IPG, _CR), jnp.int32),
          pltpu.VMEM((_IPG, _CR), jnp.int32),
          pltpu.VMEM((_IPG, _CR), jnp.int32),
          pltpu.VMEM((_NB, _CR, _HH), jnp.float32),
          pltpu.VMEM_SHARED((_NP, _HH), jnp.float32),
      ] + [pltpu.SemaphoreType.DMA] * 12,
      compiler_params=pltpu.CompilerParams(use_tc_tiling_on_sc=False),
  )
  return k(xs, src2, dst2, zeros_blk)


def _elu(y):
  return jnp.where(y > 0.0, y, jnp.exp(jnp.minimum(y, 0.0)) - 1.0)


def _mlp_in_kernel(x_ref, w_ref, b_ref, o_ref):
  y = jnp.dot(x_ref[...], w_ref[...], preferred_element_type=jnp.float32)
  o_ref[...] = _elu(y + b_ref[...])


def _gin_kernel(x_ref, p_ref, wa_ref, ba_ref, wb_ref, bb_ref,
                y_ref, s_ref, q_ref):
  h = 2.0 * x_ref[...] + p_ref[...]
  t = jnp.maximum(jnp.dot(h, wa_ref[...], preferred_element_type=jnp.float32)
                  + ba_ref[...], 0.0)
  y = jnp.dot(t, wb_ref[...], preferred_element_type=jnp.float32) + bb_ref[...]
  y = _elu(y)
  y_ref[...] = y
  s_ref[...] = jnp.sum(y, axis=0, keepdims=True).reshape(1, 1, _H)
  q_ref[...] = jnp.sum(y * y, axis=0, keepdims=True).reshape(1, 1, _H)


def _ln_apply_kernel(y_ref, m_ref, s_ref, o_ref):
  o_ref[...] = (y_ref[...] - m_ref[0, 0]) * s_ref[0, 0]


def _readout_kernel(x1_ref, x2_ref, y3_ref, m3_ref, s3_ref, b_ref,
                    wo_ref, bo_ref, wdf_ref, bdr_ref, wc_ref, bc_ref,
                    logits_ref, emb_ref, ssl_ref,
                    sums_ref, mx_ref, cnt_ref):
  i = pl.program_id(0)
  x3 = (y3_ref[...] - m3_ref[0, 0]) * s3_ref[0, 0]
  g = jnp.concatenate([x1_ref[...], x2_ref[...], x3], axis=1)
  b = b_ref[0]

  neg = jnp.float32(-3.0e38)
  srows, mrows, crows = [], [], []
  for sg in range(_NG):
    msk = b == jnp.float32(sg)
    gm = jnp.where(msk, g, 0.0)
    srows.append(jnp.sum(gm, axis=0, keepdims=True))
    gx = jnp.where(msk, g, neg)
    mrows.append(jnp.max(gx, axis=0, keepdims=True))
    crows.append(jnp.sum(msk.astype(jnp.float32), axis=0, keepdims=True))
  s_new = jnp.concatenate(srows, axis=0)
  m_new = jnp.concatenate(mrows, axis=0)
  c_new = jnp.concatenate(crows, axis=0)

  @pl.when(i == 0)
  def _():
    sums_ref[...] = s_new
    mx_ref[...] = m_new
    cnt_ref[...] = c_new

  @pl.when(i > 0)
  def _():
    sums_ref[...] += s_new
    mx_ref[...] = jnp.maximum(mx_ref[...], m_new)
    cnt_ref[...] += c_new

  @pl.when(i == _NBLK - 1)
  def _():
    mean = sums_ref[...] / jnp.maximum(cnt_ref[...], 1.0)
    r = jnp.concatenate([mean, mx_ref[...]], axis=1)
    x5 = _elu(jnp.dot(r, wo_ref[...], preferred_element_type=jnp.float32)
              + bo_ref[...])
    emb_ref[...] = x5
    sp = jnp.dot(x5, wdf_ref[...], preferred_element_type=jnp.float32) \
        + bdr_ref[...]
    ssl_ref[...] = 0.05 + 0.35 / (1.0 + jnp.exp(-sp))
    l = jnp.dot(x5, wc_ref[...], preferred_element_type=jnp.float32) \
        + bc_ref[...]
    lmax = jnp.max(l, axis=1, keepdims=True)
    lse = jnp.log(jnp.sum(jnp.exp(l - lmax), axis=1, keepdims=True)) + lmax
    logits_ref[...] = l - lse


def _row_spec():
  return pl.BlockSpec((_BLK, _H), lambda i: (i, 0))


def _full_spec(shape):
  return pl.BlockSpec(shape, lambda i: tuple(0 for _ in shape))


def _smem_spec():
  return pl.BlockSpec(memory_space=pltpu.SMEM)


def _mlp_in(x, w, b):
  return pl.pallas_call(
      _mlp_in_kernel,
      grid=(_NBLK,),
      in_specs=[_row_spec(), _full_spec((_H, _H)), _full_spec((1, _H))],
      out_specs=_row_spec(),
      out_shape=jax.ShapeDtypeStruct((_N, _H), jnp.float32),
  )(x, w, b)


def _gin_dense(x, p, wa, ba, wb, bb):
  stat = pl.BlockSpec((1, 1, _H), lambda i: (i, 0, 0))
  y, s, q = pl.pallas_call(
      _gin_kernel,
      grid=(_NBLK,),
      in_specs=[_row_spec(), _row_spec(),
                _full_spec((_H, _H)), _full_spec((1, _H)),
                _full_spec((_H, _H)), _full_spec((1, _H))],
      out_specs=[_row_spec(), stat, stat],
      out_shape=[jax.ShapeDtypeStruct((_N, _H), jnp.float32),
                 jax.ShapeDtypeStruct((_NBLK, 1, _H), jnp.float32),
                 jax.ShapeDtypeStruct((_NBLK, 1, _H), jnp.float32)],
  )(x, p, wa, ba, wb, bb)
  tot = jnp.sum(s)
  totq = jnp.sum(q)
  cnt = jnp.float32(_N * _H)
  m = tot / cnt
  v = totq / cnt - m * m
  sc = lax.rsqrt(v + 1e-5)
  return y, m.reshape(1, 1), sc.reshape(1, 1)


def _ln_apply(y, m, s):
  return pl.pallas_call(
      _ln_apply_kernel,
      grid=(_NBLK,),
      in_specs=[_row_spec(), _smem_spec(), _smem_spec()],
      out_specs=_row_spec(),
      out_shape=jax.ShapeDtypeStruct((_N, _H), jnp.float32),
  )(y, m, s)


def _readout(x1, x2, y3, m3, s3, bcol, wo, bo, wdf, bdr, wc, bc):
  outs = pl.pallas_call(
      _readout_kernel,
      grid=(_NBLK,),
      in_specs=[_row_spec(), _row_spec(), _row_spec(),
                _smem_spec(), _smem_spec(),
                pl.BlockSpec((1, _BLK, 1), lambda i: (i, 0, 0)),
                _full_spec((3 * _H * 2, _H)), _full_spec((1, _H)),
                _full_spec((_H, 3)), _full_spec((1, 3)),
                _full_spec((_H, _OUT)), _full_spec((1, _OUT))],
      out_specs=[_full_spec((_NG, _OUT)), _full_spec((_NG, _H)),
                 _full_spec((_NG, 3)), _full_spec((_NG, 3 * _H)),
                 _full_spec((_NG, 3 * _H)), _full_spec((_NG, 1))],
      out_shape=[jax.ShapeDtypeStruct((_NG, _OUT), jnp.float32),
                 jax.ShapeDtypeStruct((_NG, _H), jnp.float32),
                 jax.ShapeDtypeStruct((_NG, 3), jnp.float32),
                 jax.ShapeDtypeStruct((_NG, 3 * _H), jnp.float32),
                 jax.ShapeDtypeStruct((_NG, 3 * _H), jnp.float32),
                 jax.ShapeDtypeStruct((_NG, 1), jnp.float32)],
  )(x1, x2, y3, m3, s3, bcol, wo, bo, wdf, bdr, wc, bc)
  return outs[0], outs[1], outs[2]


def _block_diag4(w):
  z = jnp.zeros((_H, _H), jnp.float32)
  for k in range(4):
    z = z.at[32 * k:32 * (k + 1), 32 * k:32 * (k + 1)].set(w[k])
  return z


def kernel(x, edge_index, batch, W1, b1, g0Wa, g0ba, g0Wb, g0bb,
           g1Wa, g1ba, g1Wb, g1bb, Wo, bo, Wd, bd, Wc, bc):
  f32 = jnp.float32
  src = edge_index[0]
  dst = edge_index[1]
  pad = _EPAD - _E
  srcp = jnp.concatenate([src, jnp.zeros((pad,), jnp.int32)]).reshape(
      _EPAD // _CR, _CR)
  src2 = jnp.stack([srcp, srcp + _N])
  dstp = jnp.concatenate([dst, jnp.full((pad,), _N, jnp.int32)]).reshape(
      _EPAD // _CR, _CR)
  zeros_blk = jnp.zeros((_CHUNK, _HH), f32)

  b1r = b1.reshape(1, _H)
  g0bar = g0ba.reshape(1, _H)
  g0bbr = g0bb.reshape(1, _H)
  bwa = _block_diag4(g1Wa)
  bwb = _block_diag4(g1Wb)
  g1bar = g1ba.reshape(1, _H)
  g1bbr = g1bb.reshape(1, _H)
  bor = bo.reshape(1, _H)
  wdf = jnp.zeros((_H, 3), f32)
  for k in range(3):
    wdf = wdf.at[32 * k:32 * (k + 1), k].set(Wd[k, :, 0])
  bdr = bd.reshape(1, 3)
  bcr = bc.reshape(1, _OUT)
  bcol = batch.astype(f32).reshape(_NBLK, _BLK, 1)

  x1 = _mlp_in(x, W1, b1r)

  x1s = jnp.concatenate([x1[:, :_HH], x1[:, _HH:]], axis=0)
  parts1 = _sc_scatter(x1s, src2, dstp, zeros_blk)
  agg1 = jnp.concatenate([parts1[0, :_N], parts1[1, :_N]], axis=1)

  y2, m2, s2 = _gin_dense(x1, agg1, g0Wa, g0bar, g0Wb, g0bbr)
  x2 = _ln_apply(y2, m2, s2)

  x2s = jnp.concatenate([x2[:, :_HH], x2[:, _HH:]], axis=0)
  parts2 = _sc_scatter(x2s, src2, dstp, zeros_blk)
  agg2 = jnp.concatenate([parts2[0, :_N], parts2[1, :_N]], axis=1)

  y3, m3, s3 = _gin_dense(x2, agg2, bwa, g1bar, bwb, g1bbr)

  logits, emb, ssl = _readout(x1, x2, y3, m3, s3, bcol,
                              Wo, bor, wdf, bdr, Wc, bcr)
  return (logits, emb, ssl)


# speedup vs baseline: 1.0773x; 1.0773x over previous
import functools

import jax
import jax.numpy as jnp
from jax import lax
from jax.experimental import pallas as pl
from jax.experimental.pallas import tpu as pltpu
from jax.experimental.pallas import tpu_sc as plsc

_N = 10000
_E = 320000
_H = 128
_NG = 16
_OUT = 10

_NTILES = 32
_EPAD = 327680
_CHUNK = 128
_HH = 64
_CPT = _EPAD // 16
_CR = 128
_NSC = _CPT // _CR
_NB = 8
_NGRP = _NSC // _NB
_IPG = _NB
_NP = 10240
_STRIPE = _NP // 16

_BLK = 1000
_NBLK = _N // _BLK


def _sc_body(x_hbm, src2_hbm, dst2_hbm, zeros_hbm, out_hbm,
             sidx0, sidx1, didx0, didx1, rows, acc,
             sg0, sg1, sg2, sg3, sg5, sg6, sg7, sg8,
             ss0, ss1, ss2, ss3, ss5, ss6, ss7, ss8,
             sis0, sid0, sis1, sid1):
  c = lax.axis_index("c")
  s = lax.axis_index("s")
  base = s * _NSC
  sem_g = (sg0, sg1, sg2, sg3, sg5, sg6, sg7, sg8)
  sem_s = (ss0, ss1, ss2, ss3, ss5, ss6, ss7, ss8)

  pltpu.sync_copy(zeros_hbm, rows.at[0, pl.ds(0, _CHUNK)])
  for j in range(_STRIPE // _CHUNK):
    pltpu.sync_copy(rows.at[0, pl.ds(0, _CHUNK)],
                    acc.at[pl.ds(s * _STRIPE + j * _CHUNK, _CHUNK)])
  plsc.subcore_barrier()

  pltpu.async_copy(src2_hbm.at[c, pl.ds(base, _IPG)], sidx0, sis0)
  pltpu.async_copy(dst2_hbm.at[pl.ds(base, _IPG)], didx0, sid0)
  pltpu.make_async_copy(src2_hbm.at[c, pl.ds(base, _IPG)], sidx0, sis0).wait()
  pltpu.make_async_copy(dst2_hbm.at[pl.ds(base, _IPG)], didx0, sid0).wait()
  for b in range(_NB):
    pltpu.async_copy(x_hbm.at[sidx0.at[b]], rows.at[b], sem_g[b])

  def _half(g, sidx_c, didx_c, sidx_n, didx_n, sis_n, sid_n, last):
    for b in range(_NB):
      pltpu.make_async_copy(x_hbm.at[sidx_c.at[0]],
                            rows.at[b], sem_g[b]).wait()
      pltpu.async_copy(rows.at[b], acc.at[didx_c.at[b]],
                       sem_s[b], add=True)

    @pl.when(jnp.logical_not(last))
    def _():
      pltpu.async_copy(src2_hbm.at[c, pl.ds(base + (g + 1) * _IPG, _IPG)],
                       sidx_n, sis_n)
      pltpu.async_copy(dst2_hbm.at[pl.ds(base + (g + 1) * _IPG, _IPG)],
                       didx_n, sid_n)

    for b in range(_NB):
      pltpu.make_async_copy(rows.at[b], acc.at[didx_c.at[0]],
                            sem_s[b]).wait()
      if b == 0:
        @pl.when(jnp.logical_not(last))
        def _():
          pltpu.make_async_copy(
              src2_hbm.at[c, pl.ds(base, _IPG)], sidx_n, sis_n).wait()
          pltpu.make_async_copy(
              dst2_hbm.at[pl.ds(base, _IPG)], didx_n, sid_n).wait()

      @pl.when(jnp.logical_not(last))
      def _():
        pltpu.async_copy(x_hbm.at[sidx_n.at[b]],
                         rows.at[b], sem_g[b])

  def body(t, carry):
    g0 = 2 * t
    _half(g0, sidx0, didx0, sidx1, didx1, sis1, sid1, g0 >= _NGRP - 1)
    _half(g0 + 1, sidx1, didx1, sidx0, didx0, sis0, sid0, g0 + 1 >= _NGRP - 1)
    return carry

  lax.fori_loop(0, _NGRP // 2, body, 0)
  plsc.subcore_barrier()

  pltpu.sync_copy(acc.at[pl.ds(s * _STRIPE, _STRIPE)],
                  out_hbm.at[c, pl.ds(s * _STRIPE, _STRIPE)])


def _sc_scatter(xs, src2, dst2, zeros_blk):
  k = pl.kernel(
      _sc_body,
      out_type=jax.ShapeDtypeStruct((2, _NP, _HH), jnp.float32),
      mesh=plsc.VectorSubcoreMesh(core_axis_name="c", subcore_axis_name="s"),
      scratch_types=[
          pltpu.VMEM((_IPG, _CR), jnp.int32),
          pltpu.VMEM((_IPG, _CR), jnp.int32),
          pltpu.VMEM((_IPG, _CR), jnp.int32),
          pltpu.VMEM((_IPG, _CR), jnp.int32),
          pltpu.VMEM((_NB, _CR, _HH), jnp.float32),
          pltpu.VMEM_SHARED((_NP, _HH), jnp.float32),
      ] + [pltpu.SemaphoreType.DMA] * 20,
      compiler_params=pltpu.CompilerParams(use_tc_tiling_on_sc=False),
  )
  return k(xs, src2, dst2, zeros_blk)


def _elu(y):
  return jnp.where(y > 0.0, y, jnp.exp(jnp.minimum(y, 0.0)) - 1.0)


def _mlp_in_kernel(x_ref, w_ref, b_ref, o_ref):
  y = jnp.dot(x_ref[...], w_ref[...], preferred_element_type=jnp.float32)
  o_ref[...] = _elu(y + b_ref[...])


def _gin_kernel(x_ref, p_ref, wa_ref, ba_ref, wb_ref, bb_ref,
                y_ref, s_ref, q_ref):
  h = 2.0 * x_ref[...] + p_ref[...]
  t = jnp.maximum(jnp.dot(h, wa_ref[...], preferred_element_type=jnp.float32)
                  + ba_ref[...], 0.0)
  y = jnp.dot(t, wb_ref[...], preferred_element_type=jnp.float32) + bb_ref[...]
  y = _elu(y)
  y_ref[...] = y
  s_ref[...] = jnp.sum(y, axis=0, keepdims=True).reshape(1, 1, _H)
  q_ref[...] = jnp.sum(y * y, axis=0, keepdims=True).reshape(1, 1, _H)


def _ln_apply_kernel(y_ref, m_ref, s_ref, o_ref):
  o_ref[...] = (y_ref[...] - m_ref[0, 0]) * s_ref[0, 0]


def _readout_kernel(x1_ref, x2_ref, y3_ref, m3_ref, s3_ref, b_ref,
                    wo_ref, bo_ref, wdf_ref, bdr_ref, wc_ref, bc_ref,
                    logits_ref, emb_ref, ssl_ref,
                    sums_ref, mx_ref, cnt_ref):
  i = pl.program_id(0)
  x3 = (y3_ref[...] - m3_ref[0, 0]) * s3_ref[0, 0]
  g = jnp.concatenate([x1_ref[...], x2_ref[...], x3], axis=1)
  b = b_ref[0]

  neg = jnp.float32(-3.0e38)
  srows, mrows, crows = [], [], []
  for sg in range(_NG):
    msk = b == jnp.float32(sg)
    gm = jnp.where(msk, g, 0.0)
    srows.append(jnp.sum(gm, axis=0, keepdims=True))
    gx = jnp.where(msk, g, neg)
    mrows.append(jnp.max(gx, axis=0, keepdims=True))
    crows.append(jnp.sum(msk.astype(jnp.float32), axis=0, keepdims=True))
  s_new = jnp.concatenate(srows, axis=0)
  m_new = jnp.concatenate(mrows, axis=0)
  c_new = jnp.concatenate(crows, axis=0)

  @pl.when(i == 0)
  def _():
    sums_ref[...] = s_new
    mx_ref[...] = m_new
    cnt_ref[...] = c_new

  @pl.when(i > 0)
  def _():
    sums_ref[...] += s_new
    mx_ref[...] = jnp.maximum(mx_ref[...], m_new)
    cnt_ref[...] += c_new

  @pl.when(i == _NBLK - 1)
  def _():
    mean = sums_ref[...] / jnp.maximum(cnt_ref[...], 1.0)
    r = jnp.concatenate([mean, mx_ref[...]], axis=1)
    x5 = _elu(jnp.dot(r, wo_ref[...], preferred_element_type=jnp.float32)
              + bo_ref[...])
    emb_ref[...] = x5
    sp = jnp.dot(x5, wdf_ref[...], preferred_element_type=jnp.float32) \
        + bdr_ref[...]
    ssl_ref[...] = 0.05 + 0.35 / (1.0 + jnp.exp(-sp))
    l = jnp.dot(x5, wc_ref[...], preferred_element_type=jnp.float32) \
        + bc_ref[...]
    lmax = jnp.max(l, axis=1, keepdims=True)
    lse = jnp.log(jnp.sum(jnp.exp(l - lmax), axis=1, keepdims=True)) + lmax
    logits_ref[...] = l - lse


def _row_spec():
  return pl.BlockSpec((_BLK, _H), lambda i: (i, 0))


def _full_spec(shape):
  return pl.BlockSpec(shape, lambda i: tuple(0 for _ in shape))


def _smem_spec():
  return pl.BlockSpec(memory_space=pltpu.SMEM)


def _mlp_in(x, w, b):
  return pl.pallas_call(
      _mlp_in_kernel,
      grid=(_NBLK,),
      in_specs=[_row_spec(), _full_spec((_H, _H)), _full_spec((1, _H))],
      out_specs=_row_spec(),
      out_shape=jax.ShapeDtypeStruct((_N, _H), jnp.float32),
  )(x, w, b)


def _gin_dense(x, p, wa, ba, wb, bb):
  stat = pl.BlockSpec((1, 1, _H), lambda i: (i, 0, 0))
  y, s, q = pl.pallas_call(
      _gin_kernel,
      grid=(_NBLK,),
      in_specs=[_row_spec(), _row_spec(),
                _full_spec((_H, _H)), _full_spec((1, _H)),
                _full_spec((_H, _H)), _full_spec((1, _H))],
      out_specs=[_row_spec(), stat, stat],
      out_shape=[jax.ShapeDtypeStruct((_N, _H), jnp.float32),
                 jax.ShapeDtypeStruct((_NBLK, 1, _H), jnp.float32),
                 jax.ShapeDtypeStruct((_NBLK, 1, _H), jnp.float32)],
  )(x, p, wa, ba, wb, bb)
  tot = jnp.sum(s)
  totq = jnp.sum(q)
  cnt = jnp.float32(_N * _H)
  m = tot / cnt
  v = totq / cnt - m * m
  sc = lax.rsqrt(v + 1e-5)
  return y, m.reshape(1, 1), sc.reshape(1, 1)


def _ln_apply(y, m, s):
  return pl.pallas_call(
      _ln_apply_kernel,
      grid=(_NBLK,),
      in_specs=[_row_spec(), _smem_spec(), _smem_spec()],
      out_specs=_row_spec(),
      out_shape=jax.ShapeDtypeStruct((_N, _H), jnp.float32),
  )(y, m, s)


def _readout(x1, x2, y3, m3, s3, bcol, wo, bo, wdf, bdr, wc, bc):
  outs = pl.pallas_call(
      _readout_kernel,
      grid=(_NBLK,),
      in_specs=[_row_spec(), _row_spec(), _row_spec(),
                _smem_spec(), _smem_spec(),
                pl.BlockSpec((1, _BLK, 1), lambda i: (i, 0, 0)),
                _full_spec((3 * _H * 2, _H)), _full_spec((1, _H)),
                _full_spec((_H, 3)), _full_spec((1, 3)),
                _full_spec((_H, _OUT)), _full_spec((1, _OUT))],
      out_specs=[_full_spec((_NG, _OUT)), _full_spec((_NG, _H)),
                 _full_spec((_NG, 3)), _full_spec((_NG, 3 * _H)),
                 _full_spec((_NG, 3 * _H)), _full_spec((_NG, 1))],
      out_shape=[jax.ShapeDtypeStruct((_NG, _OUT), jnp.float32),
                 jax.ShapeDtypeStruct((_NG, _H), jnp.float32),
                 jax.ShapeDtypeStruct((_NG, 3), jnp.float32),
                 jax.ShapeDtypeStruct((_NG, 3 * _H), jnp.float32),
                 jax.ShapeDtypeStruct((_NG, 3 * _H), jnp.float32),
                 jax.ShapeDtypeStruct((_NG, 1), jnp.float32)],
  )(x1, x2, y3, m3, s3, bcol, wo, bo, wdf, bdr, wc, bc)
  return outs[0], outs[1], outs[2]


def _block_diag4(w):
  z = jnp.zeros((_H, _H), jnp.float32)
  for k in range(4):
    z = z.at[32 * k:32 * (k + 1), 32 * k:32 * (k + 1)].set(w[k])
  return z


def kernel(x, edge_index, batch, W1, b1, g0Wa, g0ba, g0Wb, g0bb,
           g1Wa, g1ba, g1Wb, g1bb, Wo, bo, Wd, bd, Wc, bc):
  f32 = jnp.float32
  src = edge_index[0]
  dst = edge_index[1]
  pad = _EPAD - _E
  srcp = jnp.concatenate([src, jnp.zeros((pad,), jnp.int32)]).reshape(
      _EPAD // _CR, _CR)
  src2 = jnp.stack([srcp, srcp + _N])
  dstp = jnp.concatenate([dst, jnp.full((pad,), _N, jnp.int32)]).reshape(
      _EPAD // _CR, _CR)
  zeros_blk = jnp.zeros((_CHUNK, _HH), f32)

  b1r = b1.reshape(1, _H)
  g0bar = g0ba.reshape(1, _H)
  g0bbr = g0bb.reshape(1, _H)
  bwa = _block_diag4(g1Wa)
  bwb = _block_diag4(g1Wb)
  g1bar = g1ba.reshape(1, _H)
  g1bbr = g1bb.reshape(1, _H)
  bor = bo.reshape(1, _H)
  wdf = jnp.zeros((_H, 3), f32)
  for k in range(3):
    wdf = wdf.at[32 * k:32 * (k + 1), k].set(Wd[k, :, 0])
  bdr = bd.reshape(1, 3)
  bcr = bc.reshape(1, _OUT)
  bcol = batch.astype(f32).reshape(_NBLK, _BLK, 1)

  x1 = _mlp_in(x, W1, b1r)

  x1s = jnp.concatenate([x1[:, :_HH], x1[:, _HH:]], axis=0)
  parts1 = _sc_scatter(x1s, src2, dstp, zeros_blk)
  agg1 = jnp.concatenate([parts1[0, :_N], parts1[1, :_N]], axis=1)

  y2, m2, s2 = _gin_dense(x1, agg1, g0Wa, g0bar, g0Wb, g0bbr)
  x2 = _ln_apply(y2, m2, s2)

  x2s = jnp.concatenate([x2[:, :_HH], x2[:, _HH:]], axis=0)
  parts2 = _sc_scatter(x2s, src2, dstp, zeros_blk)
  agg2 = jnp.concatenate([parts2[0, :_N], parts2[1, :_N]], axis=1)

  y3, m3, s3 = _gin_dense(x2, agg2, bwa, g1bar, bwb, g1bbr)

  logits, emb, ssl = _readout(x1, x2, y3, m3, s3, bcol,
                              Wo, bor, wdf, bdr, Wc, bcr)
  return (logits, emb, ssl)


# speedup vs baseline: 1.0941x; 1.0156x over previous
import functools

import jax
import jax.numpy as jnp
from jax import lax
from jax.experimental import pallas as pl
from jax.experimental.pallas import tpu as pltpu
from jax.experimental.pallas import tpu_sc as plsc

_N = 10000
_E = 320000
_H = 128
_NG = 16
_OUT = 10

_NTILES = 32
_EPAD = 327680
_CHUNK = 128
_HH = 64
_CPT = _EPAD // 16
_CR = 128
_NSC = _CPT // _CR
_NB = 8
_NGRP = _NSC // _NB
_IPG = _NB
_NP = 10240
_STRIPE = _NP // 16

_BLK = 1000
_NBLK = _N // _BLK


def _sc_body(x_hbm, src2_hbm, dst2_hbm, zeros_hbm, out_hbm,
             sidx0, sidx1, didx0, didx1, rows, acc,
             sg0, sg1, sg2, sg3, sg5, sg6, sg7, sg8,
             ss0, ss1, ss2, ss3, ss5, ss6, ss7, ss8,
             sis0, sid0, sis1, sid1):
  c = lax.axis_index("c")
  s = lax.axis_index("s")
  base = s * _NSC
  sem_g = (sg0, sg1, sg2, sg3, sg5, sg6, sg7, sg8)
  sem_s = (ss0, ss1, ss2, ss3, ss5, ss6, ss7, ss8)

  pltpu.sync_copy(zeros_hbm, rows.at[0, pl.ds(0, _CHUNK)])
  for j in range(_STRIPE // _CHUNK):
    pltpu.sync_copy(rows.at[0, pl.ds(0, _CHUNK)],
                    acc.at[pl.ds(s * _STRIPE + j * _CHUNK, _CHUNK)])
  plsc.subcore_barrier()

  pltpu.async_copy(src2_hbm.at[c, pl.ds(base, _IPG)], sidx0, sis0)
  pltpu.async_copy(dst2_hbm.at[pl.ds(base, _IPG)], didx0, sid0)
  pltpu.make_async_copy(src2_hbm.at[c, pl.ds(base, _IPG)], sidx0, sis0).wait()
  pltpu.make_async_copy(dst2_hbm.at[pl.ds(base, _IPG)], didx0, sid0).wait()
  for b in range(_NB):
    pltpu.async_copy(x_hbm.at[sidx0.at[b]], rows.at[b], sem_g[b])

  def _half(g, sidx_c, didx_c, sidx_n, didx_n, sis_n, sid_n, last):
    for b in range(_NB):
      pltpu.make_async_copy(x_hbm.at[sidx_c.at[0]],
                            rows.at[b], sem_g[b]).wait()
      pltpu.async_copy(rows.at[b], acc.at[didx_c.at[b]],
                       sem_s[b], add=True)

    @pl.when(jnp.logical_not(last))
    def _():
      pltpu.async_copy(src2_hbm.at[c, pl.ds(base + (g + 1) * _IPG, _IPG)],
                       sidx_n, sis_n)
      pltpu.async_copy(dst2_hbm.at[pl.ds(base + (g + 1) * _IPG, _IPG)],
                       didx_n, sid_n)

    for b in range(_NB):
      pltpu.make_async_copy(rows.at[b], acc.at[didx_c.at[0]],
                            sem_s[b]).wait()
      if b == 0:
        @pl.when(jnp.logical_not(last))
        def _():
          pltpu.make_async_copy(
              src2_hbm.at[c, pl.ds(base, _IPG)], sidx_n, sis_n).wait()
          pltpu.make_async_copy(
              dst2_hbm.at[pl.ds(base, _IPG)], didx_n, sid_n).wait()

      @pl.when(jnp.logical_not(last))
      def _():
        pltpu.async_copy(x_hbm.at[sidx_n.at[b]],
                         rows.at[b], sem_g[b])

  def body(t, carry):
    g0 = 2 * t
    _half(g0, sidx0, didx0, sidx1, didx1, sis1, sid1, g0 >= _NGRP - 1)
    _half(g0 + 1, sidx1, didx1, sidx0, didx0, sis0, sid0, g0 + 1 >= _NGRP - 1)
    return carry

  lax.fori_loop(0, _NGRP // 2, body, 0)
  plsc.subcore_barrier()

  pltpu.sync_copy(acc.at[pl.ds(s * _STRIPE, _STRIPE)],
                  out_hbm.at[c, pl.ds(s * _STRIPE, _STRIPE)])


def _sc_scatter(xs, src2, dst2, zeros_blk):
  k = pl.kernel(
      _sc_body,
      out_type=jax.ShapeDtypeStruct((2, _NP, _HH), jnp.float32),
      mesh=plsc.VectorSubcoreMesh(core_axis_name="c", subcore_axis_name="s"),
      scratch_types=[
          pltpu.VMEM((_IPG, _CR), jnp.int32),
          pltpu.VMEM((_IPG, _CR), jnp.int32),
          pltpu.VMEM((_IPG, _CR), jnp.int32),
          pltpu.VMEM((_IPG, _CR), jnp.int32),
          pltpu.VMEM((_NB, _CR, _HH), jnp.float32),
          pltpu.VMEM_SHARED((_NP, _HH), jnp.float32),
      ] + [pltpu.SemaphoreType.DMA] * 20,
      compiler_params=pltpu.CompilerParams(use_tc_tiling_on_sc=False),
  )
  return k(xs, src2, dst2, zeros_blk)


def _elu(y):
  return jnp.where(y > 0.0, y, jnp.exp(jnp.minimum(y, 0.0)) - 1.0)


def _split_store(o_ref, y):
  o_ref[0] = y[:, :_HH]
  o_ref[1] = y[:, _HH:]


def _mlp_in_kernel(x_ref, w_ref, b_ref, o_ref):
  y = jnp.dot(x_ref[...], w_ref[...], preferred_element_type=jnp.float32)
  _split_store(o_ref, _elu(y + b_ref[...]))


def _gin_kernel(x_ref, p_ref, wa_ref, ba_ref, wb_ref, bb_ref,
                y_ref, s_ref, q_ref):
  x = jnp.concatenate([x_ref[0], x_ref[1]], axis=1)
  p = jnp.concatenate([p_ref[0], p_ref[1]], axis=1)
  h = 2.0 * x + p
  t = jnp.maximum(jnp.dot(h, wa_ref[...], preferred_element_type=jnp.float32)
                  + ba_ref[...], 0.0)
  y = jnp.dot(t, wb_ref[...], preferred_element_type=jnp.float32) + bb_ref[...]
  y = _elu(y)
  _split_store(y_ref, y)
  s_ref[...] = jnp.sum(y, axis=0, keepdims=True).reshape(1, 1, _H)
  q_ref[...] = jnp.sum(y * y, axis=0, keepdims=True).reshape(1, 1, _H)


def _ln_apply_kernel(y_ref, m_ref, s_ref, o_ref):
  o_ref[...] = (y_ref[...] - m_ref[0, 0]) * s_ref[0, 0]


def _readout_kernel(x1_ref, x2_ref, y3_ref, m3_ref, s3_ref, bc_ref, br_ref,
                    wo_ref, bo_ref, wdf_ref, bdr_ref, wc_ref, bc2_ref,
                    logits_ref, emb_ref, ssl_ref,
                    sums_ref, mx_ref, cnt_ref):
  i = pl.program_id(0)
  m3 = m3_ref[0, 0]
  s3 = s3_ref[0, 0]
  g = jnp.concatenate(
      [x1_ref[0], x1_ref[1], x2_ref[0], x2_ref[1],
       (y3_ref[0] - m3) * s3, (y3_ref[1] - m3) * s3], axis=1)
  bcol = bc_ref[0]
  brow = br_ref[0]

  seg_iota = lax.broadcasted_iota(jnp.int32, (_NG, _BLK), 0).astype(jnp.float32)
  mask = (seg_iota == brow).astype(jnp.float32)
  s_new = jnp.dot(mask, g, preferred_element_type=jnp.float32)
  c_new = jnp.sum(mask, axis=1, keepdims=True)

  @pl.when(i == 0)
  def _():
    sums_ref[...] = jnp.zeros((_NG, 3 * _H), jnp.float32)
    mx_ref[...] = jnp.full((_NG, 3 * _H), -3.0e38, jnp.float32)
    cnt_ref[...] = jnp.zeros((_NG, 1), jnp.float32)

  sums_ref[...] += s_new
  cnt_ref[...] += c_new

  bmin = jnp.min(bcol)
  bmax = jnp.max(bcol)
  for sg in range(_NG):
    @pl.when((jnp.float32(sg) >= bmin) & (jnp.float32(sg) <= bmax))
    def _():
      gx = jnp.where(bcol == jnp.float32(sg), g, -3.0e38)
      mx_ref[sg:sg + 1, :] = jnp.maximum(
          mx_ref[sg:sg + 1, :], jnp.max(gx, axis=0, keepdims=True))

  @pl.when(i == _NBLK - 1)
  def _():
    mean = sums_ref[...] / jnp.maximum(cnt_ref[...], 1.0)
    r = jnp.concatenate([mean, mx_ref[...]], axis=1)
    x5 = _elu(jnp.dot(r, wo_ref[...], preferred_element_type=jnp.float32)
              + bo_ref[...])
    emb_ref[...] = x5
    sp = jnp.dot(x5, wdf_ref[...], preferred_element_type=jnp.float32) \
        + bdr_ref[...]
    ssl_ref[...] = 0.05 + 0.35 / (1.0 + jnp.exp(-sp))
    l = jnp.dot(x5, wc_ref[...], preferred_element_type=jnp.float32) \
        + bc2_ref[...]
    lmax = jnp.max(l, axis=1, keepdims=True)
    lse = jnp.log(jnp.sum(jnp.exp(l - lmax), axis=1, keepdims=True)) + lmax
    logits_ref[...] = l - lse


def _row_spec():
  return pl.BlockSpec((_BLK, _H), lambda i: (i, 0))


def _half_spec():
  return pl.BlockSpec((2, _BLK, _HH), lambda i: (0, i, 0))


def _full_spec(shape):
  return pl.BlockSpec(shape, lambda i: tuple(0 for _ in shape))


def _smem_spec():
  return pl.BlockSpec(memory_space=pltpu.SMEM)


def _mlp_in(x, w, b):
  return pl.pallas_call(
      _mlp_in_kernel,
      grid=(_NBLK,),
      in_specs=[_row_spec(), _full_spec((_H, _H)), _full_spec((1, _H))],
      out_specs=_half_spec(),
      out_shape=jax.ShapeDtypeStruct((2, _N, _HH), jnp.float32),
  )(x, w, b)


def _gin_dense(xs, parts, wa, ba, wb, bb):
  stat = pl.BlockSpec((1, 1, _H), lambda i: (i, 0, 0))
  y, s, q = pl.pallas_call(
      _gin_kernel,
      grid=(_NBLK,),
      in_specs=[_half_spec(), _half_spec(),
                _full_spec((_H, _H)), _full_spec((1, _H)),
                _full_spec((_H, _H)), _full_spec((1, _H))],
      out_specs=[_half_spec(), stat, stat],
      out_shape=[jax.ShapeDtypeStruct((2, _N, _HH), jnp.float32),
                 jax.ShapeDtypeStruct((_NBLK, 1, _H), jnp.float32),
                 jax.ShapeDtypeStruct((_NBLK, 1, _H), jnp.float32)],
  )(xs, parts, wa, ba, wb, bb)
  tot = jnp.sum(s)
  totq = jnp.sum(q)
  cnt = jnp.float32(_N * _H)
  m = tot / cnt
  v = totq / cnt - m * m
  sc = lax.rsqrt(v + 1e-5)
  return y, m.reshape(1, 1), sc.reshape(1, 1)


def _ln_apply(ys, m, s):
  blk = pl.BlockSpec((1, _BLK, _HH), lambda i, j: (j, i, 0))
  return pl.pallas_call(
      _ln_apply_kernel,
      grid=(_NBLK, 2),
      in_specs=[blk, _smem_spec(), _smem_spec()],
      out_specs=blk,
      out_shape=jax.ShapeDtypeStruct((2, _N, _HH), jnp.float32),
  )(ys, m, s)


def _readout(x1s, x2s, y3s, m3, s3, bcol, brow, wo, bo, wdf, bdr, wc, bc):
  outs = pl.pallas_call(
      _readout_kernel,
      grid=(_NBLK,),
      in_specs=[_half_spec(), _half_spec(), _half_spec(),
                _smem_spec(), _smem_spec(),
                pl.BlockSpec((1, _BLK, 1), lambda i: (i, 0, 0)),
                pl.BlockSpec((1, 1, _BLK), lambda i: (i, 0, 0)),
                _full_spec((3 * _H * 2, _H)), _full_spec((1, _H)),
                _full_spec((_H, 3)), _full_spec((1, 3)),
                _full_spec((_H, _OUT)), _full_spec((1, _OUT))],
      out_specs=[_full_spec((_NG, _OUT)), _full_spec((_NG, _H)),
                 _full_spec((_NG, 3)), _full_spec((_NG, 3 * _H)),
                 _full_spec((_NG, 3 * _H)), _full_spec((_NG, 1))],
      out_shape=[jax.ShapeDtypeStruct((_NG, _OUT), jnp.float32),
                 jax.ShapeDtypeStruct((_NG, _H), jnp.float32),
                 jax.ShapeDtypeStruct((_NG, 3), jnp.float32),
                 jax.ShapeDtypeStruct((_NG, 3 * _H), jnp.float32),
                 jax.ShapeDtypeStruct((_NG, 3 * _H), jnp.float32),
                 jax.ShapeDtypeStruct((_NG, 1), jnp.float32)],
  )(x1s, x2s, y3s, m3, s3, bcol, brow, wo, bo, wdf, bdr, wc, bc)
  return outs[0], outs[1], outs[2]


def _block_diag4(w):
  z = jnp.zeros((_H, _H), jnp.float32)
  for k in range(4):
    z = z.at[32 * k:32 * (k + 1), 32 * k:32 * (k + 1)].set(w[k])
  return z


def kernel(x, edge_index, batch, W1, b1, g0Wa, g0ba, g0Wb, g0bb,
           g1Wa, g1ba, g1Wb, g1bb, Wo, bo, Wd, bd, Wc, bc):
  f32 = jnp.float32
  src = edge_index[0]
  dst = edge_index[1]
  pad = _EPAD - _E
  srcp = jnp.concatenate([src, jnp.zeros((pad,), jnp.int32)]).reshape(
      _EPAD // _CR, _CR)
  src2 = jnp.stack([srcp, srcp + _N])
  dstp = jnp.concatenate([dst, jnp.full((pad,), _N, jnp.int32)]).reshape(
      _EPAD // _CR, _CR)
  zeros_blk = jnp.zeros((_CHUNK, _HH), f32)

  b1r = b1.reshape(1, _H)
  g0bar = g0ba.reshape(1, _H)
  g0bbr = g0bb.reshape(1, _H)
  bwa = _block_diag4(g1Wa)
  bwb = _block_diag4(g1Wb)
  g1bar = g1ba.reshape(1, _H)
  g1bbr = g1bb.reshape(1, _H)
  bor = bo.reshape(1, _H)
  wdf = jnp.zeros((_H, 3), f32)
  for k in range(3):
    wdf = wdf.at[32 * k:32 * (k + 1), k].set(Wd[k, :, 0])
  bdr = bd.reshape(1, 3)
  bcr = bc.reshape(1, _OUT)
  bcol = batch.astype(f32).reshape(_NBLK, _BLK, 1)
  brow = batch.astype(f32).reshape(_NBLK, 1, _BLK)

  x1s = _mlp_in(x, W1, b1r)

  parts1 = _sc_scatter(x1s.reshape(2 * _N, _HH), src2, dstp, zeros_blk)

  y2s, m2, s2 = _gin_dense(x1s, parts1, g0Wa, g0bar, g0Wb, g0bbr)
  x2s = _ln_apply(y2s, m2, s2)

  parts2 = _sc_scatter(x2s.reshape(2 * _N, _HH), src2, dstp, zeros_blk)

  y3s, m3, s3 = _gin_dense(x2s, parts2, bwa, g1bar, bwb, g1bbr)

  logits, emb, ssl = _readout(x1s, x2s, y3s, m3, s3, bcol, brow,
                              Wo, bor, wdf, bdr, Wc, bcr)
  return (logits, emb, ssl)
